# Initial kernel scaffold; baseline (speedup 1.0000x reference)
import jax
import jax.numpy as jnp
from jax.experimental import pallas as pl


def kernel(i, j, table_i, table_j):
    raise NotImplementedError("write your pallas kernel here")


# speedup vs baseline: 5.3864x; 5.3864x over previous
import jax
import jax.numpy as jnp
from jax.experimental import pallas as pl
from jax.experimental.pallas import tpu as pltpu

_B, _H, _W = 4, 224, 224
_T = 224
_C = 128
_HB = 8
_N = _HB * _W
_GRID = _B * (_H // _HB)


def _kernel(i_ref, j_ref, ti_ref, tj_ref, out_ref):
    idx_i = i_ref[0, 0, :]
    idx_j = j_ref[0, 0, :]
    rows = jax.lax.broadcasted_iota(jnp.int32, (_T, _N), 0)
    onehot_i = (idx_i[None, :] == rows).astype(jnp.float32)
    onehot_j = (idx_j[None, :] == rows).astype(jnp.float32)
    out_ref[0, :_C, :] = jnp.dot(ti_ref[...], onehot_i,
                                 preferred_element_type=jnp.float32)
    out_ref[0, _C:, :] = jnp.dot(tj_ref[...], onehot_j,
                                 preferred_element_type=jnp.float32)


def kernel(i, j, table_i, table_j):
    ti_t = table_i.T
    tj_t = table_j.T
    i_r = i.reshape(_GRID, 1, _N).astype(jnp.int32)
    j_r = j.reshape(_GRID, 1, _N).astype(jnp.int32)
    nhb = _H // _HB

    out = pl.pallas_call(
        _kernel,
        grid=(_GRID,),
        in_specs=[
            pl.BlockSpec((1, 1, _N), lambda g: (g, 0, 0)),
            pl.BlockSpec((1, 1, _N), lambda g: (g, 0, 0)),
            pl.BlockSpec((_C, _T), lambda g: (0, 0)),
            pl.BlockSpec((_C, _T), lambda g: (0, 0)),
        ],
        out_specs=pl.BlockSpec((1, 2 * _C, _N), lambda g: (g // nhb, 0, g % nhb)),
        out_shape=jax.ShapeDtypeStruct((_B, 2 * _C, _H * _W), jnp.float32),
        compiler_params=pltpu.CompilerParams(
            dimension_semantics=("arbitrary",),
        ),
    )(i_r, j_r, ti_t, tj_t)
    return out.reshape(_B, 2 * _C, _H, _W)
